# Initial kernel scaffold; baseline (speedup 1.0000x reference)
import jax
import jax.numpy as jnp
from jax.experimental import pallas as pl


def kernel(x_user, x_item, edge_index_u2i, edge_index_i2u, edge_attr_u2i, edge_attr_i2u, W1_u2i, b1_u2i, W2_u2i, b2_u2i, W1_i2u, b1_i2u, W2_i2u, b2_i2u, Wu_user, bu_user, g_user, be_user, Wu_item, bu_item, g_item, be_item):
    raise NotImplementedError("write your pallas kernel here")


# speedup vs baseline: 1.0210x; 1.0210x over previous
import functools

import jax
import jax.numpy as jnp
from jax.experimental import pallas as pl
from jax.experimental.pallas import tpu as pltpu

N = 50000
D = 128
E = 500000

ROW_BLOCK = 2000


def _edge_mlp_body(x_ref, w1_ref, b1_ref, w2_ref, b2_ref, o_ref):
    h = jnp.maximum(
        jnp.dot(x_ref[...], w1_ref[...], preferred_element_type=jnp.float32)
        + b1_ref[...],
        0.0,
    )
    o_ref[...] = (
        jnp.dot(h, w2_ref[...], preferred_element_type=jnp.float32) + b2_ref[...]
    )


def _edge_mlp(x, w1, b1, w2, b2):
    grid = N // ROW_BLOCK
    return pl.pallas_call(
        _edge_mlp_body,
        grid=(grid,),
        in_specs=[
            pl.BlockSpec((ROW_BLOCK, D), lambda i: (i, 0)),
            pl.BlockSpec((D, D), lambda i: (0, 0)),
            pl.BlockSpec((D,), lambda i: (0,)),
            pl.BlockSpec((D, D), lambda i: (0, 0)),
            pl.BlockSpec((D,), lambda i: (0,)),
        ],
        out_specs=pl.BlockSpec((ROW_BLOCK, D), lambda i: (i, 0)),
        out_shape=jax.ShapeDtypeStruct((N, D), jnp.float32),
    )(x, w1, b1, w2, b2)


def _node_update_body(aggr_ref, x_ref, wu_ref, bu_ref, g_ref, be_ref, o_ref):
    h = aggr_ref[...] + x_ref[...]
    h = jnp.dot(h, wu_ref[...], preferred_element_type=jnp.float32) + bu_ref[...]
    mu = jnp.mean(h, axis=-1, keepdims=True)
    var = jnp.mean((h - mu) ** 2, axis=-1, keepdims=True)
    h = (h - mu) * jax.lax.rsqrt(var + 1e-5) * g_ref[...] + be_ref[...]
    o_ref[...] = jnp.maximum(h, 0.0)


def _node_update(aggr, x, wu, bu, g, be):
    grid = N // ROW_BLOCK
    return pl.pallas_call(
        _node_update_body,
        grid=(grid,),
        in_specs=[
            pl.BlockSpec((ROW_BLOCK, D), lambda i: (i, 0)),
            pl.BlockSpec((ROW_BLOCK, D), lambda i: (i, 0)),
            pl.BlockSpec((D, D), lambda i: (0, 0)),
            pl.BlockSpec((D,), lambda i: (0,)),
            pl.BlockSpec((D,), lambda i: (0,)),
            pl.BlockSpec((D,), lambda i: (0,)),
        ],
        out_specs=pl.BlockSpec((ROW_BLOCK, D), lambda i: (i, 0)),
        out_shape=jax.ShapeDtypeStruct((N, D), jnp.float32),
    )(aggr, x, wu, bu, g, be)


def kernel(x_user, x_item, edge_index_u2i, edge_index_i2u, edge_attr_u2i,
           edge_attr_i2u, W1_u2i, b1_u2i, W2_u2i, b2_u2i, W1_i2u, b1_i2u,
           W2_i2u, b2_i2u, Wu_user, bu_user, g_user, be_user, Wu_item,
           bu_item, g_item, be_item):
    t_u2i = _edge_mlp(x_user, W1_u2i, b1_u2i, W2_u2i, b2_u2i)
    t_i2u = _edge_mlp(x_item, W1_i2u, b1_i2u, W2_i2u, b2_i2u)

    msg_u2i = jnp.take(t_u2i, edge_index_u2i[0], axis=0) * edge_attr_u2i[:, None]
    aggr_item = jax.ops.segment_sum(msg_u2i, edge_index_u2i[1], num_segments=N)
    msg_i2u = jnp.take(t_i2u, edge_index_i2u[0], axis=0) * edge_attr_i2u[:, None]
    aggr_user = jax.ops.segment_sum(msg_i2u, edge_index_i2u[1], num_segments=N)

    out_user = _node_update(aggr_user, x_user, Wu_user, bu_user, g_user, be_user)
    out_item = _node_update(aggr_item, x_item, Wu_item, bu_item, g_item, be_item)
    return (out_user, out_item)
